# Initial kernel scaffold; baseline (speedup 1.0000x reference)
import jax
import jax.numpy as jnp
from jax.experimental import pallas as pl


def kernel(query, key, value, julia_c_real, julia_c_imag, escape_scale):
    raise NotImplementedError("write your pallas kernel here")


# speedup vs baseline: 1.6371x; 1.6371x over previous
import functools
import math

import jax
import jax.numpy as jnp
from jax.experimental import pallas as pl
from jax.experimental.pallas import tpu as pltpu

JULIA_ITERS = 64
ESCAPE_RADIUS = 2.0

BQ = 1024
BK = 512


def _julia_weight_kernel(cr_ref, ci_ref, es_ref, x_ref, w_ref):
    x = x_ref[...]
    cr = cr_ref[0]
    ci = ci_ref[0]
    es = es_ref[0]
    zr = x
    zi = jnp.zeros_like(x)
    escaped = jnp.zeros(x.shape, dtype=jnp.bool_)
    et = jnp.ones_like(x)
    r2 = ESCAPE_RADIUS * ESCAPE_RADIUS
    for it in range(JULIA_ITERS):
        nzr = zr * zr - zi * zi + cr
        nzi = 2.0 * zr * zi + ci
        zr = jnp.where(escaped, zr, nzr)
        zi = jnp.where(escaped, zi, nzi)
        mag2 = zr * zr + zi * zi
        newly = jnp.logical_and(jnp.logical_not(escaped), mag2 > r2)
        et = jnp.where(newly, jnp.float32(it / JULIA_ITERS), et)
        escaped = jnp.logical_or(escaped, newly)
    w_ref[...] = jnp.exp(et * es) + 1e-8


def _flash_kernel(q_ref, k_ref, vt_ref, w_ref, o_ref, acc_ref, m_ref):
    ki = pl.program_id(2)
    nk = pl.num_programs(2)

    @pl.when(ki == 0)
    def _init():
        m_ref[...] = jnp.full_like(m_ref, -1e30)
        acc_ref[...] = jnp.zeros_like(acc_ref)

    q = q_ref[0] * jnp.float32(1.0 / math.sqrt(64.0))
    k = k_ref[0]
    st = jax.lax.dot_general(k, q, (((1,), (1,)), ((), ())),
                             preferred_element_type=jnp.float32)

    m_prev = m_ref[...]
    m_cur = jnp.max(st, axis=0, keepdims=True)
    m_next = jnp.maximum(m_prev, m_cur)
    alpha = jnp.exp(m_prev - m_next)
    pt = jnp.exp(st - m_next)

    u = vt_ref[0] * w_ref[...]
    acc_ref[...] = acc_ref[...] * alpha + jax.lax.dot_general(
        u, pt, (((1,), (0,)), ((), ())),
        preferred_element_type=jnp.float32)
    m_ref[...] = m_next

    @pl.when(ki == nk - 1)
    def _done():
        acc = acc_ref[...]
        o_ref[0] = acc[:64, :] * (1.0 / acc[64:65, :])


@functools.partial(jax.jit, static_argnames=())
def kernel(query, key, value, julia_c_real, julia_c_imag, escape_scale):
    B, S, D = key.shape
    x = jnp.linspace(-2.0, 2.0, S).reshape(8, S // 8)

    w = pl.pallas_call(
        _julia_weight_kernel,
        out_shape=jax.ShapeDtypeStruct((8, S // 8), jnp.float32),
        in_specs=[
            pl.BlockSpec(memory_space=pltpu.SMEM),
            pl.BlockSpec(memory_space=pltpu.SMEM),
            pl.BlockSpec(memory_space=pltpu.SMEM),
            pl.BlockSpec((8, S // 8), lambda: (0, 0)),
        ],
        out_specs=pl.BlockSpec((8, S // 8), lambda: (0, 0)),
    )(
        julia_c_real.reshape(1),
        julia_c_imag.reshape(1),
        escape_scale.reshape(1),
        x,
    )
    w_row = w.reshape(1, S)

    vt = jnp.swapaxes(value, 1, 2)
    vt_aug = jnp.concatenate(
        [vt, jnp.ones((B, 1, S), dtype=vt.dtype)], axis=1)

    nq = S // BQ
    nk = S // BK
    out_t = pl.pallas_call(
        _flash_kernel,
        grid=(B, nq, nk),
        in_specs=[
            pl.BlockSpec((1, BQ, D), lambda b, qi, ki: (b, qi, 0)),
            pl.BlockSpec((1, BK, D), lambda b, qi, ki: (b, ki, 0)),
            pl.BlockSpec((1, D + 1, BK), lambda b, qi, ki: (b, 0, ki)),
            pl.BlockSpec((1, BK), lambda b, qi, ki: (0, ki)),
        ],
        out_specs=pl.BlockSpec((1, D, BQ), lambda b, qi, ki: (b, 0, qi)),
        out_shape=jax.ShapeDtypeStruct((B, D, S), jnp.float32),
        scratch_shapes=[
            pltpu.VMEM((D + 1, BQ), jnp.float32),
            pltpu.VMEM((1, BQ), jnp.float32),
        ],
        compiler_params=pltpu.CompilerParams(
            dimension_semantics=("parallel", "parallel", "arbitrary"),
        ),
    )(query, key, vt_aug, w_row)

    return jnp.swapaxes(out_t, 1, 2)
